# Initial kernel scaffold; baseline (speedup 1.0000x reference)
import jax
import jax.numpy as jnp
from jax.experimental import pallas as pl


def kernel(code, centroid):
    raise NotImplementedError("write your pallas kernel here")


# speedup vs baseline: 4.3405x; 4.3405x over previous
import functools

import jax
import jax.numpy as jnp
from jax import lax
from jax.experimental import pallas as pl
from jax.experimental.pallas import tpu as pltpu
from jax.experimental.pallas import tpu_sc as plsc

NUM_SUB = 8
K = 1024
SUB_DIM = 64
C = 65536

NC = 2
NS = 16
L = 16
NW = NC * NS

B = C * NUM_SUB
ROWS_W = B // NW
CHUNK = 128
NCHUNK = ROWS_W // CHUNK


@functools.partial(
    pl.kernel,
    out_type=jax.ShapeDtypeStruct((B, SUB_DIM), jnp.float32),
    mesh=plsc.VectorSubcoreMesh(
        core_axis_name="c", subcore_axis_name="s", num_cores=NC, num_subcores=NS
    ),
    scratch_types=[
        pltpu.VMEM((CHUNK,), jnp.int32),
        pltpu.VMEM((CHUNK, SUB_DIM), jnp.float32),
        pltpu.SemaphoreType.DMA,
    ],
    compiler_params=pltpu.CompilerParams(use_tc_tiling_on_sc=False),
)
def _pq_decode(code_hbm, table_hbm, out_hbm, idx_v, rows_v, sem):
    wid = lax.axis_index("s") * NC + lax.axis_index("c")
    base = wid * ROWS_W
    off = (lax.rem(lax.iota(jnp.int32, L), jnp.int32(NUM_SUB))) * jnp.int32(K)

    def chunk_body(g, carry):
        rb = base + g * CHUNK
        pltpu.sync_copy(code_hbm.at[pl.ds(rb, CHUNK)], idx_v)
        for j in range(CHUNK // L):
            sl = pl.ds(j * L, L)
            idx_v[sl] = idx_v[sl] + off
        pltpu.async_copy(table_hbm.at[idx_v], rows_v, sem).wait()
        pltpu.sync_copy(rows_v, out_hbm.at[pl.ds(rb, CHUNK)])
        return carry

    lax.fori_loop(0, NCHUNK, chunk_body, 0)


def kernel(code, centroid):
    code_flat = code.reshape(B)
    table = centroid.reshape(NUM_SUB * K, SUB_DIM)
    out = _pq_decode(code_flat, table)
    return out.reshape(C, NUM_SUB * SUB_DIM)


# speedup vs baseline: 6.4428x; 1.4844x over previous
import functools

import jax
import jax.numpy as jnp
from jax import lax
from jax.experimental import pallas as pl
from jax.experimental.pallas import tpu as pltpu
from jax.experimental.pallas import tpu_sc as plsc

NUM_SUB = 8
K = 1024
SUB_DIM = 64
C = 65536
DIM = NUM_SUB * SUB_DIM
PAD_DIM = 128

NC = 2
NS = 16
L = 16
NW = NC * NS

B = C * NUM_SUB
ROWS_W = B // NW
CHUNK = 128
PAIR = 2 * CHUNK
NPAIR = ROWS_W // PAIR
OUT_W = ROWS_W // NUM_SUB
OUT_P = PAIR // NUM_SUB


def _repack(g_f32, st_f32):
    for j in range(2):
        def row_body(c, carry, j=j):
            for u in range(DIM // L):
                r = c * NUM_SUB + u // 4
                st_f32[j * (CHUNK // NUM_SUB) + c, pl.ds(u * L, L)] = (
                    g_f32[j, r, pl.ds((u % 4) * L, L)]
                )
            return carry

        lax.fori_loop(0, CHUNK // NUM_SUB, row_body, 0)


@functools.partial(
    pl.kernel,
    out_type=jax.ShapeDtypeStruct((C, DIM), jnp.float32),
    mesh=plsc.VectorSubcoreMesh(
        core_axis_name="c", subcore_axis_name="s", num_cores=NC, num_subcores=NS
    ),
    scratch_types=[
        pltpu.VMEM((ROWS_W,), jnp.int32),
        pltpu.VMEM((2, CHUNK, PAD_DIM), jnp.float32),
        pltpu.VMEM((2, CHUNK, PAD_DIM), jnp.float32),
        pltpu.VMEM((OUT_P, DIM), jnp.float32),
        pltpu.VMEM((OUT_P, DIM), jnp.float32),
        pltpu.SemaphoreType.DMA,
        pltpu.SemaphoreType.DMA,
    ],
)
def _pq_decode(code_hbm, table_hbm, out_hbm, idx_v, ga, gb, sta, stb, gsem, wsem):
    wid = lax.axis_index("s") * NC + lax.axis_index("c")
    base = wid * ROWS_W
    out_base = wid * OUT_W

    pltpu.sync_copy(code_hbm.at[pl.ds(base, ROWS_W)], idx_v)
    off = lax.rem(lax.iota(jnp.int32, L), jnp.int32(NUM_SUB)) * jnp.int32(K)

    def add_body(i, carry):
        sl = pl.ds(i * L, L)
        idx_v[sl] = idx_v[sl] + off
        return carry

    lax.fori_loop(0, ROWS_W // L, add_body, 0)

    def fire_gathers(p, gbuf):
        for j in range(2):
            idx = idx_v.at[pl.ds(p * PAIR + j * CHUNK, CHUNK)]
            pltpu.async_copy(table_hbm.at[idx], gbuf.at[j], gsem)

    def drain_gathers(gbuf):
        for j in range(2):
            pltpu.make_async_copy(table_hbm.at[idx_v.at[pl.ds(0, CHUNK)]],
                                  gbuf.at[j], gsem).wait()

    def fire_write(p, stbuf):
        pltpu.async_copy(stbuf, out_hbm.at[pl.ds(out_base + p * OUT_P, OUT_P)],
                         wsem)

    def drain_write(stbuf):
        pltpu.make_async_copy(stbuf, out_hbm.at[pl.ds(out_base, OUT_P)],
                              wsem).wait()

    fire_gathers(0, ga)
    fire_gathers(1, gb)
    drain_gathers(ga)
    _repack(ga, sta)
    fire_write(0, sta)
    fire_gathers(2, ga)
    drain_gathers(gb)
    _repack(gb, stb)
    fire_write(1, stb)

    def body(u, carry):
        p0 = 2 * u
        fire_gathers(p0 + 1, gb)
        drain_gathers(ga)
        drain_write(sta)
        _repack(ga, sta)
        fire_write(p0, sta)
        fire_gathers(p0 + 2, ga)
        drain_gathers(gb)
        drain_write(stb)
        _repack(gb, stb)
        fire_write(p0 + 1, stb)
        return carry

    lax.fori_loop(1, NPAIR // 2 - 1, body, 0)

    fire_gathers(NPAIR - 1, gb)
    drain_gathers(ga)
    drain_write(sta)
    _repack(ga, sta)
    fire_write(NPAIR - 2, sta)
    drain_gathers(gb)
    drain_write(stb)
    _repack(gb, stb)
    fire_write(NPAIR - 1, stb)
    drain_write(sta)
    drain_write(stb)


def kernel(code, centroid):
    code_flat = code.reshape(B)
    table = jnp.pad(
        centroid.reshape(NUM_SUB * K, SUB_DIM),
        ((0, 0), (0, PAD_DIM - SUB_DIM)),
    )
    return _pq_decode(code_flat, table)
